# Initial kernel scaffold; baseline (speedup 1.0000x reference)
import jax
import jax.numpy as jnp
from jax.experimental import pallas as pl


def kernel(x, edge_attr, params, edge_index, batch_node, batch_edge):
    raise NotImplementedError("write your pallas kernel here")


# speedup vs baseline: 1.8302x; 1.8302x over previous
import functools

import jax
import jax.numpy as jnp
from jax import lax
from jax.experimental import pallas as pl
from jax.experimental.pallas import tpu as pltpu
from jax.experimental.pallas import tpu_sc as plsc

F32 = jnp.float32
I32 = jnp.int32
EPS = 1e-5
NG = 8

SC_CORES = 2
SC_SUBCORES = 16
SC_WORKERS = SC_CORES * SC_SUBCORES
SC_CHUNK = 128


def _mr_from_stats(st_ref, n):
    s1 = st_ref[0, :]
    s2 = st_ref[1, :]
    m = s1 / n
    var = (s2 - n * m * m) / (n - 1)
    return m, 1.0 / jnp.sqrt(var + EPS)


def _apply_act(y, act):
    if act == "relu":
        return jnp.maximum(y, 0.0)
    if act == "leaky":
        return jnp.where(y >= 0, y, 0.01 * y)
    if act == "sigmoid":
        return 1.0 / (1.0 + jnp.exp(-y))
    return y


def _gpass(E, blk, specs, W, b, act, want_stats=False, seg_be=None,
           out_rows=None, mm_bf16=True):
    arrs = []
    in_specs = []

    def add(a, spec):
        arrs.append(a)
        in_specs.append(spec)
        return len(arrs) - 1

    def rows(w):
        return pl.BlockSpec((blk, w), lambda i: (i, 0))

    def full8(w):
        return pl.BlockSpec((8, w), lambda i: (0, 0))

    loaders = []
    for sp in specs:
        k = sp["kind"]
        if k == "plain":
            w = sp["a"].shape[1]
            loaders.append((k, {"a": add(sp["a"], rows(w))}, None))
        elif k == "norm":
            w = sp["a"].shape[1]
            sl = {"a": add(sp["a"], rows(w)),
                  "st": add(sp["stats"], full8(w))}
            loaders.append((k, sl, sp["n"]))
        elif k == "norm_scale":
            w = sp["a"].shape[1]
            sl = {"a": add(sp["a"], rows(w)),
                  "st": add(sp["stats"], full8(w)),
                  "tab": add(sp["tab"], full8(sp["tab"].shape[1])),
                  "be": add(sp["be"], rows(1))}
            loaders.append((k, sl, sp["n"]))
        elif k == "aggmean":
            w = sp["p0"].shape[1]
            sl = {"p0": add(sp["p0"], rows(w)), "p1": add(sp["p1"], rows(w)),
                  "c0": add(sp["c0"], rows(w)), "c1": add(sp["c1"], rows(w)),
                  "st": add(sp["stats"], full8(w))}
            loaders.append((k, sl, sp["n"]))
        else:
            raise ValueError(k)

    widths = []
    for sp in specs:
        a = sp.get("a", sp.get("p0"))
        widths.append(a.shape[1])
    n_out = W.shape[1] if W is not None else sum(widths)

    w_slot = b_slot = seg_slot = None
    if W is not None:
        w_slot = add(W, pl.BlockSpec(W.shape, lambda i: (0, 0)))
        b_slot = add(b.reshape(1, -1), pl.BlockSpec((1, n_out),
                                                    lambda i: (0, 0)))
    if seg_be is not None:
        seg_slot = add(seg_be, rows(1))

    R = out_rows if out_rows is not None else E
    out_shape = [jax.ShapeDtypeStruct((R, n_out), F32)]
    out_specs = [rows(n_out)]
    if want_stats:
        out_shape.append(jax.ShapeDtypeStruct((8, n_out), F32))
        out_specs.append(full8(n_out))
    if seg_be is not None:
        for _ in range(4):
            out_shape.append(jax.ShapeDtypeStruct((8, n_out), F32))
            out_specs.append(full8(n_out))

    n_in = len(arrs)

    def body(*refs):
        ins = refs[:n_in]
        outs = refs[n_in:]
        i = pl.program_id(0)
        parts = []
        for (k, sl, n) in loaders:
            if k == "plain":
                v = ins[sl["a"]][...]
            elif k == "norm":
                m, r = _mr_from_stats(ins[sl["st"]], n)
                v = (ins[sl["a"]][...] - m) * r
            elif k == "norm_scale":
                m, r = _mr_from_stats(ins[sl["st"]], n)
                v = (ins[sl["a"]][...] - m) * r
                bev = ins[sl["be"]][...]
                oh = (bev == lax.broadcasted_iota(I32, (blk, NG), 1)
                      ).astype(F32)
                v = v * jnp.dot(oh, ins[sl["tab"]][...],
                                preferred_element_type=F32,
                                precision=lax.Precision.HIGHEST)
            else:
                m, r = _mr_from_stats(ins[sl["st"]], n)
                s = ins[sl["p0"]][...] + ins[sl["p1"]][...]
                c = ins[sl["c0"]][...] + ins[sl["c1"]][...]
                mean = s / jnp.maximum(c, 1.0)
                v = (mean - jnp.where(c > 0, m, 0.0)) * r
            parts.append(v)
        xcat = parts[0] if len(parts) == 1 else jnp.concatenate(parts, axis=1)
        if W is not None:
            if mm_bf16:
                y = jnp.dot(xcat.astype(jnp.bfloat16),
                            ins[w_slot][...].astype(jnp.bfloat16),
                            preferred_element_type=F32)
            else:
                y = jnp.dot(xcat, ins[w_slot][...],
                            preferred_element_type=F32,
                            precision=lax.Precision.HIGHEST)
            y = y + ins[b_slot][...]
        else:
            y = xcat
        y = _apply_act(y, act)
        outs[0][...] = y
        oi = 1
        if want_stats:
            st = jnp.concatenate(
                [jnp.sum(y, axis=0, keepdims=True),
                 jnp.sum(y * y, axis=0, keepdims=True),
                 jnp.zeros((6, n_out), F32)], axis=0)
            sref = outs[oi]
            oi += 1

            @pl.when(i == 0)
            def _():
                sref[...] = st

            @pl.when(i != 0)
            def _():
                sref[...] = sref[...] + st

        if seg_be is not None:
            bev = ins[seg_slot][...]
            oh = (bev == lax.broadcasted_iota(I32, (blk, NG), 1)).astype(F32)
            ssum = lax.dot_general(oh, y, (((0,), (0,)), ((), ())),
                                   preferred_element_type=F32,
                                   precision=lax.Precision.HIGHEST)
            scnt = jnp.sum(oh, axis=0)[:, None] + jnp.zeros((NG, n_out), F32)
            smax = jnp.concatenate(
                [jnp.max(jnp.where(bev == g, y, -jnp.inf), axis=0,
                         keepdims=True) for g in range(NG)], axis=0)
            smin = jnp.concatenate(
                [jnp.min(jnp.where(bev == g, y, jnp.inf), axis=0,
                         keepdims=True) for g in range(NG)], axis=0)
            refs4 = outs[oi:oi + 4]
            vals4 = (ssum, smax, smin, scnt)
            ops4 = (jnp.add, jnp.maximum, jnp.minimum, jnp.add)

            @pl.when(i == 0)
            def _():
                for rf, vv in zip(refs4, vals4):
                    rf[...] = vv

            @pl.when(i != 0)
            def _():
                for rf, vv, op in zip(refs4, vals4, ops4):
                    rf[...] = op(rf[...], vv)

    res = pl.pallas_call(
        body,
        grid=(E // blk,),
        in_specs=in_specs,
        out_specs=out_specs,
        out_shape=out_shape,
    )(*arrs)
    return res


def _wc_mlp(segs, stats, n, p):
    ssum, smax, smin, scnt = segs
    ops = [ssum, smax, smin, scnt, stats,
           p["l1"]["w"], p["l1"]["b"].reshape(1, -1),
           p["n1"]["g"].reshape(1, -1), p["n1"]["b"].reshape(1, -1),
           p["l2"]["w"], p["l2"]["b"].reshape(1, -1),
           p["n2"]["g"].reshape(1, -1), p["n2"]["b"].reshape(1, -1)]

    def fullspec(a):
        return pl.BlockSpec(a.shape, lambda i: tuple(0 for _ in a.shape))

    d_out = p["l2"]["w"].shape[1]

    def body(ssum_r, smax_r, smin_r, scnt_r, st_r, w1_r, b1_r, g1_r, gb1_r,
             w2_r, b2_r, g2_r, gb2_r, out_r):
        m, r = _mr_from_stats(st_r, n)
        cnt = scnt_r[...]
        mean = ssum_r[...] / jnp.maximum(cnt, 1.0)
        em = (mean - jnp.where(cnt > 0, m, 0.0)) * r
        emx = (smax_r[...] - m) * r
        emn = (smin_r[...] - m) * r
        h = jnp.concatenate([em, emx, emn], axis=1)
        h = jnp.dot(h.astype(jnp.bfloat16), w1_r[...].astype(jnp.bfloat16),
                    preferred_element_type=F32) + b1_r[...]
        h = jnp.where(h >= 0, h, 0.01 * h)
        mu = jnp.mean(h, axis=-1, keepdims=True)
        va = jnp.mean((h - mu) ** 2, axis=-1, keepdims=True)
        h = (h - mu) * 1.0 / jnp.sqrt(va + EPS) * g1_r[...] + gb1_r[...]
        h = jnp.dot(h.astype(jnp.bfloat16), w2_r[...].astype(jnp.bfloat16),
                    preferred_element_type=F32) + b2_r[...]
        h = jnp.where(h >= 0, h, 0.01 * h)
        mu = jnp.mean(h, axis=-1, keepdims=True)
        va = jnp.mean((h - mu) ** 2, axis=-1, keepdims=True)
        out_r[...] = (h - mu) * 1.0 / jnp.sqrt(va + EPS) * g2_r[...] + gb2_r[...]

    return pl.pallas_call(
        body,
        grid=(1,),
        in_specs=[fullspec(a) for a in ops],
        out_specs=pl.BlockSpec((NG, d_out), lambda i: (0, 0)),
        out_shape=jax.ShapeDtypeStruct((NG, d_out), F32),
    )(*ops)


def _pair_mean(eo2, blk=2000):
    M = eo2.shape[0]

    def body(a_ref, o_ref):
        v = a_ref[...]
        o_ref[...] = 0.5 * (v[:, :1] + v[:, 1:2])

    return pl.pallas_call(
        body,
        grid=(M // blk,),
        in_specs=[pl.BlockSpec((blk, 2), lambda i: (i, 0))],
        out_specs=pl.BlockSpec((blk, 1), lambda i: (i, 0)),
        out_shape=jax.ShapeDtypeStruct((M, 1), F32),
    )(eo2)


def _sc_gather(table, row_pad, col_pad):
    EPAD = row_pad.shape[0]
    D = table.shape[1]
    per_w = EPAD // SC_WORKERS
    n_ch = per_w // SC_CHUNK
    mesh = plsc.VectorSubcoreMesh(core_axis_name="c", subcore_axis_name="s")

    @functools.partial(
        pl.kernel, mesh=mesh,
        compiler_params=pltpu.CompilerParams(use_tc_tiling_on_sc=False),
        out_type=[jax.ShapeDtypeStruct((EPAD, D), F32),
                  jax.ShapeDtypeStruct((EPAD, D), F32)],
        scratch_types=[
            pltpu.VMEM((SC_CHUNK,), I32), pltpu.VMEM((SC_CHUNK, D), F32),
            pltpu.VMEM((SC_CHUNK,), I32), pltpu.VMEM((SC_CHUNK, D), F32),
            pltpu.SemaphoreType.DMA, pltpu.SemaphoreType.DMA,
        ])
    def k(table_h, row_h, col_h, src_h, dst_h,
          idx_a, rows_a, idx_b, rows_b, sem_a, sem_b):
        wid = lax.axis_index("s") * SC_CORES + lax.axis_index("c")
        base = wid * per_w

        def body(j, carry):
            off = base + j * SC_CHUNK
            pltpu.sync_copy(row_h.at[pl.ds(off, SC_CHUNK)], idx_a)
            pltpu.sync_copy(col_h.at[pl.ds(off, SC_CHUNK)], idx_b)
            ca = pltpu.async_copy(table_h.at[idx_a], rows_a, sem_a)
            cb = pltpu.async_copy(table_h.at[idx_b], rows_b, sem_b)
            ca.wait()
            cb.wait()
            pltpu.sync_copy(rows_a, src_h.at[pl.ds(off, SC_CHUNK)])
            pltpu.sync_copy(rows_b, dst_h.at[pl.ds(off, SC_CHUNK)])
            return carry

        lax.fori_loop(0, n_ch, body, 0)

    return k(table, row_pad, col_pad)


def _sc_scatter_one(vals, col_pad, zeros_acc, ones_chunk, N, count_mode):
    EPAD = col_pad.shape[0]
    D = zeros_acc.shape[1]
    NACC = zeros_acc.shape[0]
    per_w = EPAD // SC_WORKERS
    n_ch = per_w // SC_CHUNK
    cp = (N // SC_SUBCORES + 7) // 8 * 8
    last = N - cp * (SC_SUBCORES - 1)
    mesh = plsc.VectorSubcoreMesh(core_axis_name="c", subcore_axis_name="s")

    @functools.partial(
        pl.kernel, mesh=mesh,
        compiler_params=pltpu.CompilerParams(use_tc_tiling_on_sc=False),
        out_type=jax.ShapeDtypeStruct((2 * N, D), F32),
        scratch_types=[
            pltpu.VMEM((SC_CHUNK,), I32), pltpu.VMEM((SC_CHUNK, D), F32),
            pltpu.VMEM_SHARED((NACC, D), F32),
        ])
    def k(vals_h, col_h, zeros_h, sum_h, idx_v, rows_v, acc_s):
        c = lax.axis_index("c")
        s = lax.axis_index("s")

        @pl.when(s == 0)
        def _():
            pltpu.sync_copy(zeros_h, acc_s)

        plsc.subcore_barrier()
        if count_mode:
            pltpu.sync_copy(vals_h, rows_v)
        base = (c * SC_SUBCORES + s) * per_w

        def body(j, carry):
            off = base + j * SC_CHUNK
            pltpu.sync_copy(col_h.at[pl.ds(off, SC_CHUNK)], idx_v)
            if not count_mode:
                pltpu.sync_copy(vals_h.at[pl.ds(off, SC_CHUNK)], rows_v)
            pltpu.sync_copy(rows_v, acc_s.at[idx_v], add=True)
            return carry

        lax.fori_loop(0, n_ch, body, 0)
        plsc.subcore_barrier()

        obase = c * N

        @pl.when(s < SC_SUBCORES - 1)
        def _():
            start = s * cp
            pltpu.sync_copy(acc_s.at[pl.ds(start, cp)],
                            sum_h.at[pl.ds(obase + start, cp)])

        @pl.when(s == SC_SUBCORES - 1)
        def _():
            start = (SC_SUBCORES - 1) * cp
            pltpu.sync_copy(acc_s.at[pl.ds(start, last)],
                            sum_h.at[pl.ds(obase + start, last)])

    return k(ones_chunk if count_mode else vals, col_pad, zeros_acc)


def _sc_scatter(vals, col_pad, zeros_acc, ones_chunk, N):
    sums = _sc_scatter_one(vals, col_pad, zeros_acc, ones_chunk, N, False)
    cnts = _sc_scatter_one(vals, col_pad, zeros_acc, ones_chunk, N, True)
    return sums, cnts


def kernel(x, edge_attr, params, edge_index, batch_node, batch_edge):
    E = edge_attr.shape[0]
    N = x.shape[0]
    EB = 2000
    NB = 2000
    EPAD = -(-E // (SC_WORKERS * SC_CHUNK)) * (SC_WORKERS * SC_CHUNK)
    NACC = N + 8

    be = batch_edge.reshape(E, 1).astype(I32)
    bn = batch_node.reshape(N, 1).astype(I32)
    row = edge_index[0].astype(I32)
    col = edge_index[1].astype(I32)
    pad_g = jnp.zeros((EPAD - E,), I32)
    row_pad = jnp.concatenate([row, pad_g])
    col_pad = jnp.concatenate([col, pad_g])
    col_pad_s = jnp.concatenate([col, jnp.full((EPAD - E,), N, I32)])

    def plain(a):
        return {"kind": "plain", "a": a}

    def norm(a, st, n):
        return {"kind": "norm", "a": a, "stats": st, "n": n}

    def norm_scale(a, st, n, tab, b_):
        return {"kind": "norm_scale", "a": a, "stats": st, "n": n,
                "tab": tab, "be": b_}

    enc = params["enc_edge"]
    h1, h1_st = _gpass(E, EB, [plain(edge_attr)], enc[0]["w"], enc[0]["b"],
                       "relu", want_stats=True, mm_bf16=False)
    ea0, ea0_st, e_ss, e_sx, e_sn, e_sc = _gpass(
        E, EB, [norm(h1, h1_st, E)], enc[1]["w"], enc[1]["b"], "relu",
        want_stats=True, seg_be=be)

    c0 = params["cores"][0]
    wE0 = _wc_mlp((e_ss, e_sx, e_sn, e_sc), ea0_st, E, params["wEdgeC"])
    w1a = c0["edge_mlp1"][0]
    u1, u1_st = _gpass(E, EB, [norm_scale(ea0, ea0_st, E, wE0, be)],
                       w1a["w"][32:48], w1a["b"], "relu", want_stats=True)
    w1b = c0["edge_mlp1"][1]
    e1, e1_st = _gpass(E, EB, [norm(u1, u1_st, E)], w1b["w"], w1b["b"],
                       "relu", want_stats=True)
    w2a = c0["edge_mlp2"][0]
    b1, b1_st = _gpass(E, EB, [norm(e1, e1_st, E),
                               norm_scale(ea0, ea0_st, E, wE0, be)],
                       w2a["w"], w2a["b"], "relu", want_stats=True)
    w2b = c0["edge_mlp2"][1]
    ea1, ea1_st, e1_ss, e1_sx, e1_sn, e1_sc = _gpass(
        E, EB, [norm(b1, b1_st, E)], w2b["w"], w2b["b"], "relu",
        want_stats=True, seg_be=be)
    n1a = c0["node_mlp1"][0]
    o1, o1_st = _gpass(E, EB, [norm(ea1, ea1_st, E)], n1a["w"][16:32],
                       n1a["b"], "relu", want_stats=True)
    n1b = c0["node_mlp1"][1]
    out0, out0_st = _gpass(E, EB, [norm(o1, o1_st, E)], n1b["w"], n1b["b"],
                           "relu", want_stats=True, out_rows=EPAD)

    zeros_acc = jnp.zeros((NACC, 16), F32)
    ones_chunk = jnp.ones((SC_CHUNK, 16), F32)
    sums2, cnts2 = _sc_scatter(out0, col_pad_s, zeros_acc, ones_chunk, N)
    p0, p1 = sums2[:N], sums2[N:]
    q0, q1 = cnts2[:N], cnts2[N:]

    n2a = c0["node_mlp2"][0]
    hh, hh_st = _gpass(N, NB, [{"kind": "aggmean", "p0": p0, "p1": p1,
                                "c0": q0, "c1": q1, "stats": out0_st,
                                "n": E}],
                       n2a["w"][16:32], n2a["b"], "relu", want_stats=True)
    n2b = c0["node_mlp2"][1]
    x2, x2_st, n_ss, n_sx, n_sn, n_sc = _gpass(
        N, NB, [norm(hh, hh_st, N)], n2b["w"], n2b["b"], "relu",
        want_stats=True, seg_be=bn)

    c1 = params["cores"][1]
    wN1 = _wc_mlp((n_ss, n_sx, n_sn, n_sc), x2_st, N, params["wNodeC"])
    wE1 = _wc_mlp((e1_ss, e1_sx, e1_sn, e1_sc), ea1_st, E, params["wEdgeC"])
    (xs,) = _gpass(N, NB, [norm_scale(x2, x2_st, N, wN1, bn)], None, None,
                   None)
    src_p, dst_p = _sc_gather(xs, row_pad, col_pad)
    src, dst = src_p[:E], dst_p[:E]

    m1a = c1["edge_mlp1"][0]
    u1b, u1b_st = _gpass(E, EB, [plain(src), plain(dst),
                                 norm_scale(ea1, ea1_st, E, wE1, be)],
                         m1a["w"], m1a["b"], "relu", want_stats=True)
    m1b = c1["edge_mlp1"][1]
    e1b, e1b_st = _gpass(E, EB, [norm(u1b, u1b_st, E)], m1b["w"], m1b["b"],
                         "relu", want_stats=True)
    m2a = c1["edge_mlp2"][0]
    b1b, b1b_st = _gpass(E, EB, [norm(e1b, e1b_st, E),
                                 norm_scale(ea1, ea1_st, E, wE1, be)],
                         m2a["w"], m2a["b"], "relu", want_stats=True)
    m2b = c1["edge_mlp2"][1]
    ea2, ea2_st = _gpass(E, EB, [norm(b1b, b1b_st, E)], m2b["w"], m2b["b"],
                         "relu", want_stats=True)

    d0, d1 = params["dec_edge"]
    t1, t1_st = _gpass(E, EB, [norm(ea2, ea2_st, E)], d0["w"], d0["b"],
                       "relu", want_stats=True, mm_bf16=False)
    (eo,) = _gpass(E, EB, [norm(t1, t1_st, E)], d1["w"], d1["b"], "sigmoid")
    return _pair_mean(eo.reshape(E // 2, 2))
